# Initial kernel scaffold; baseline (speedup 1.0000x reference)
import jax
import jax.numpy as jnp
from jax.experimental import pallas as pl


def kernel(data, length, embed, W, b):
    raise NotImplementedError("write your pallas kernel here")


# speedup vs baseline: 13.6715x; 13.6715x over previous
import functools

import jax
import jax.numpy as jnp
from jax import lax
from jax.experimental import pallas as pl
from jax.experimental.pallas import tpu as pltpu
from jax.experimental.pallas import tpu_sc as plsc

B = 16384
L = 200
EMB = 32
NCLS = 16

NC = 2
NS = 16
NW = NC * NS
D_TILE = B // NW
HALF = L // 2
NCHUNK = 2
DC = D_TILE // NCHUNK


def _sc_pool(idx_hr, embed):
    mesh = plsc.VectorSubcoreMesh(core_axis_name="c", subcore_axis_name="s")

    @functools.partial(
        pl.kernel,
        mesh=mesh,
        out_type=jax.ShapeDtypeStruct((B, EMB), jnp.float32),
        compiler_params=pltpu.CompilerParams(use_tc_tiling_on_sc=False),
        scratch_types=[
            pltpu.VMEM((DC * 2, HALF), jnp.int32),
            pltpu.VMEM((2, L, EMB), jnp.float32),
            pltpu.VMEM((D_TILE, EMB), jnp.float32),
            pltpu.SemaphoreType.DMA((2,)),
        ],
    )
    def k(idx_hbm, embed_hbm, out_hbm, idx_v, bufs, out_v, gsem):
        wid = lax.axis_index("s") * NC + lax.axis_index("c")
        hr_base = wid * (D_TILE * 2)

        def fire(d, par):
            pltpu.async_copy(
                embed_hbm.at[idx_v.at[2 * d]],
                bufs.at[par].at[pl.ds(0, HALF)],
                gsem.at[par],
            )
            pltpu.async_copy(
                embed_hbm.at[idx_v.at[2 * d + 1]],
                bufs.at[par].at[pl.ds(HALF, HALF)],
                gsem.at[par],
            )

        def wait_par(par):
            pltpu.make_async_copy(
                embed_hbm.at[idx_v.at[0]],
                bufs.at[par].at[pl.ds(0, HALF)],
                gsem.at[par],
            ).wait()
            pltpu.make_async_copy(
                embed_hbm.at[idx_v.at[0]],
                bufs.at[par].at[pl.ds(HALF, HALF)],
                gsem.at[par],
            ).wait()

        def sum_rows(buf):
            def body(i, accs):
                a0, a1 = accs
                for u in range(8):
                    r = i * 8 + u
                    a0 = a0 + buf[r, pl.ds(0, 16)]
                    a1 = a1 + buf[r, pl.ds(16, 16)]
                return a0, a1

            z = jnp.zeros((16,), jnp.float32)
            return lax.fori_loop(0, L // 8, body, (z, z), unroll=False)

        def chunk_body(c, _):
            hr0 = hr_base + c * (DC * 2)
            pltpu.sync_copy(idx_hbm.at[pl.ds(hr0, DC * 2)], idx_v)
            fire(0, 0)
            fire(1, 1)

            def pair_body(p, _):
                for par in (0, 1):
                    d = 2 * p + par
                    wait_par(par)
                    a0, a1 = sum_rows(bufs.at[par])
                    nd = d + 2

                    @pl.when(nd < DC)
                    def _():
                        fire(nd, par)

                    row = c * DC + d
                    out_v[row, pl.ds(0, 16)] = a0
                    out_v[row, pl.ds(16, 16)] = a1
                return 0

            lax.fori_loop(0, DC // 2, pair_body, 0, unroll=False)
            return 0

        lax.fori_loop(0, NCHUNK, chunk_body, 0, unroll=False)
        pltpu.sync_copy(out_v, out_hbm.at[pl.ds(wid * D_TILE, D_TILE)])

    return k(idx_hr, embed)


def _tc_head(pooled, len_f, W, b2):
    def body(p_ref, l_ref, w_ref, b_ref, o_ref):
        x = p_ref[...] / l_ref[...]
        o_ref[...] = (
            lax.dot_general(
                x,
                w_ref[...],
                dimension_numbers=(((1,), (1,)), ((), ())),
                preferred_element_type=jnp.float32,
            )
            + b_ref[...]
        )

    return pl.pallas_call(
        body,
        out_shape=jax.ShapeDtypeStruct((B, NCLS), jnp.float32),
    )(pooled, len_f, W, b2)


def kernel(data, length, embed, W, b):
    idx_hr = data.astype(jnp.int32).reshape(B * 2, HALF)
    len_f = length.astype(jnp.float32).reshape(B, 1)
    pooled = _sc_pool(idx_hr, embed)
    return _tc_head(pooled, len_f, W, b.reshape(1, NCLS))


# speedup vs baseline: 16.0876x; 1.1767x over previous
import functools

import jax
import jax.numpy as jnp
from jax import lax
from jax.experimental import pallas as pl
from jax.experimental.pallas import tpu as pltpu
from jax.experimental.pallas import tpu_sc as plsc

B = 16384
L = 200
EMB = 32
NCLS = 16

NC = 2
NS = 16
NW = NC * NS
D_TILE = B // NW
HALF = L // 2
NCHUNK = 2
DC = D_TILE // NCHUNK
NBUF = 4


def _sc_pool(idx_hr, embed):
    mesh = plsc.VectorSubcoreMesh(core_axis_name="c", subcore_axis_name="s")

    @functools.partial(
        pl.kernel,
        mesh=mesh,
        out_type=jax.ShapeDtypeStruct((B, EMB), jnp.float32),
        compiler_params=pltpu.CompilerParams(use_tc_tiling_on_sc=False),
        scratch_types=[
            pltpu.VMEM((DC * 2, HALF), jnp.int32),
            pltpu.VMEM((NBUF, L, EMB), jnp.float32),
            pltpu.VMEM((D_TILE, EMB), jnp.float32),
            pltpu.SemaphoreType.DMA((NBUF,)),
        ],
    )
    def k(idx_hbm, embed_hbm, out_hbm, idx_v, bufs, out_v, gsem):
        wid = lax.axis_index("s") * NC + lax.axis_index("c")
        hr_base = wid * (D_TILE * 2)

        def fire(d, par):
            pltpu.async_copy(
                embed_hbm.at[idx_v.at[2 * d]],
                bufs.at[par].at[pl.ds(0, HALF)],
                gsem.at[par],
            )
            pltpu.async_copy(
                embed_hbm.at[idx_v.at[2 * d + 1]],
                bufs.at[par].at[pl.ds(HALF, HALF)],
                gsem.at[par],
            )

        def wait_par(par):
            pltpu.make_async_copy(
                embed_hbm.at[idx_v.at[0]],
                bufs.at[par].at[pl.ds(0, HALF)],
                gsem.at[par],
            ).wait()
            pltpu.make_async_copy(
                embed_hbm.at[idx_v.at[0]],
                bufs.at[par].at[pl.ds(HALF, HALF)],
                gsem.at[par],
            ).wait()

        def sum_rows(buf):
            def body(i, accs):
                a0, a1 = accs
                for u in range(8):
                    r = i * 8 + u
                    a0 = a0 + buf[r, pl.ds(0, 16)]
                    a1 = a1 + buf[r, pl.ds(16, 16)]
                return a0, a1

            z = jnp.zeros((16,), jnp.float32)
            return lax.fori_loop(0, L // 8, body, (z, z), unroll=False)

        def chunk_body(c, _):
            hr0 = hr_base + c * (DC * 2)
            pltpu.sync_copy(idx_hbm.at[pl.ds(hr0, DC * 2)], idx_v)
            for par in range(NBUF):
                fire(par, par)

            def group_body(p, _):
                for par in range(NBUF):
                    d = NBUF * p + par
                    wait_par(par)
                    a0, a1 = sum_rows(bufs.at[par])
                    nd = d + NBUF

                    @pl.when(nd < DC)
                    def _():
                        fire(nd, par)

                    row = c * DC + d
                    out_v[row, pl.ds(0, 16)] = a0
                    out_v[row, pl.ds(16, 16)] = a1
                return 0

            lax.fori_loop(0, DC // NBUF, group_body, 0, unroll=False)
            return 0

        lax.fori_loop(0, NCHUNK, chunk_body, 0, unroll=False)
        pltpu.sync_copy(out_v, out_hbm.at[pl.ds(wid * D_TILE, D_TILE)])

    return k(idx_hr, embed)


def _tc_head(pooled, len_f, W, b2):
    def body(p_ref, l_ref, w_ref, b_ref, o_ref):
        x = p_ref[...] / l_ref[...]
        o_ref[...] = (
            lax.dot_general(
                x,
                w_ref[...],
                dimension_numbers=(((1,), (1,)), ((), ())),
                preferred_element_type=jnp.float32,
            )
            + b_ref[...]
        )

    return pl.pallas_call(
        body,
        out_shape=jax.ShapeDtypeStruct((B, NCLS), jnp.float32),
    )(pooled, len_f, W, b2)


def kernel(data, length, embed, W, b):
    idx_hr = data.astype(jnp.int32).reshape(B * 2, HALF)
    len_f = length.astype(jnp.float32).reshape(B, 1)
    pooled = _sc_pool(idx_hr, embed)
    return _tc_head(pooled, len_f, W, b.reshape(1, NCLS))


# speedup vs baseline: 16.7587x; 1.0417x over previous
import functools

import jax
import jax.numpy as jnp
from jax import lax
from jax.experimental import pallas as pl
from jax.experimental.pallas import tpu as pltpu
from jax.experimental.pallas import tpu_sc as plsc

B = 16384
L = 200
EMB = 32
NCLS = 16

NC = 2
NS = 16
NW = NC * NS
D_TILE = B // NW
HALF = L // 2
NCHUNK = 2
DC = D_TILE // NCHUNK
NBUF = 8


def _sc_pool(idx_hr, embed):
    mesh = plsc.VectorSubcoreMesh(core_axis_name="c", subcore_axis_name="s")

    @functools.partial(
        pl.kernel,
        mesh=mesh,
        out_type=jax.ShapeDtypeStruct((B, EMB), jnp.float32),
        compiler_params=pltpu.CompilerParams(use_tc_tiling_on_sc=False),
        scratch_types=[
            pltpu.VMEM((DC * 2, HALF), jnp.int32),
            pltpu.VMEM((NBUF, L, EMB), jnp.float32),
            pltpu.VMEM((D_TILE, EMB), jnp.float32),
            pltpu.SemaphoreType.DMA((NBUF,)),
        ],
    )
    def k(idx_hbm, embed_hbm, out_hbm, idx_v, bufs, out_v, gsem):
        wid = lax.axis_index("s") * NC + lax.axis_index("c")
        hr_base = wid * (D_TILE * 2)

        def fire(d, par):
            pltpu.async_copy(
                embed_hbm.at[idx_v.at[2 * d]],
                bufs.at[par].at[pl.ds(0, HALF)],
                gsem.at[par],
            )
            pltpu.async_copy(
                embed_hbm.at[idx_v.at[2 * d + 1]],
                bufs.at[par].at[pl.ds(HALF, HALF)],
                gsem.at[par],
            )

        def wait_par(par):
            pltpu.make_async_copy(
                embed_hbm.at[idx_v.at[0]],
                bufs.at[par].at[pl.ds(0, HALF)],
                gsem.at[par],
            ).wait()
            pltpu.make_async_copy(
                embed_hbm.at[idx_v.at[0]],
                bufs.at[par].at[pl.ds(HALF, HALF)],
                gsem.at[par],
            ).wait()

        def sum_rows(buf):
            def body(i, accs):
                a0, a1 = accs
                for u in range(8):
                    r = i * 8 + u
                    a0 = a0 + buf[r, pl.ds(0, 16)]
                    a1 = a1 + buf[r, pl.ds(16, 16)]
                return a0, a1

            z = jnp.zeros((16,), jnp.float32)
            return lax.fori_loop(0, L // 8, body, (z, z), unroll=False)

        def chunk_body(c, _):
            hr0 = hr_base + c * (DC * 2)
            pltpu.sync_copy(idx_hbm.at[pl.ds(hr0, DC * 2)], idx_v)
            for par in range(NBUF):
                fire(par, par)

            def group_body(p, _):
                for par in range(NBUF):
                    d = NBUF * p + par
                    wait_par(par)
                    a0, a1 = sum_rows(bufs.at[par])
                    nd = d + NBUF

                    @pl.when(nd < DC)
                    def _():
                        fire(nd, par)

                    row = c * DC + d
                    out_v[row, pl.ds(0, 16)] = a0
                    out_v[row, pl.ds(16, 16)] = a1
                return 0

            lax.fori_loop(0, DC // NBUF, group_body, 0, unroll=False)
            return 0

        lax.fori_loop(0, NCHUNK, chunk_body, 0, unroll=False)
        pltpu.sync_copy(out_v, out_hbm.at[pl.ds(wid * D_TILE, D_TILE)])

    return k(idx_hr, embed)


def _tc_head(pooled, len_f, W, b2):
    def body(p_ref, l_ref, w_ref, b_ref, o_ref):
        x = p_ref[...] / l_ref[...]
        o_ref[...] = (
            lax.dot_general(
                x,
                w_ref[...],
                dimension_numbers=(((1,), (1,)), ((), ())),
                preferred_element_type=jnp.float32,
            )
            + b_ref[...]
        )

    return pl.pallas_call(
        body,
        out_shape=jax.ShapeDtypeStruct((B, NCLS), jnp.float32),
    )(pooled, len_f, W, b2)


def kernel(data, length, embed, W, b):
    idx_hr = data.astype(jnp.int32).reshape(B * 2, HALF)
    len_f = length.astype(jnp.float32).reshape(B, 1)
    pooled = _sc_pool(idx_hr, embed)
    return _tc_head(pooled, len_f, W, b.reshape(1, NCLS))
